# baseline (speedup 1.0000x reference)
import jax
import jax.numpy as jnp
from jax.experimental import pallas as pl

_CHUNK = 512


def _chamfer_body(p_ref, g_ref, loss_ref, p2g_ref, g2p_ref):
    b = pl.program_id(0)
    bs = pl.num_programs(0)
    pts = p_ref[0]
    gpts = g_ref[0]
    m = gpts.shape[0]
    r = jax.lax.broadcasted_iota(jnp.int32, (3, 3), 0)
    c = jax.lax.broadcasted_iota(jnp.int32, (3, 3), 1)
    eye = jnp.where(r == c, 1.0, 0.0).astype(jnp.float32)
    g = jax.lax.dot_general(
        eye, gpts, (((1,), (1,)), ((), ())),
        precision=jax.lax.Precision.HIGHEST,
        preferred_element_type=jnp.float32,
    )
    px = pts[:, 0:1]
    py = pts[:, 1:2]
    pz = pts[:, 2:3]
    rowmin = None
    g2p_sum = None
    for k in range(0, m, _CHUNK):
        gx = g[0:1, k:k + _CHUNK]
        gy = g[1:2, k:k + _CHUNK]
        gz = g[2:3, k:k + _CHUNK]
        dx = px - gx
        dy = py - gy
        dz = pz - gz
        d = dx * dx + dy * dy + dz * dz
        rm = jnp.min(d, axis=1, keepdims=True)
        rowmin = rm if rowmin is None else jnp.minimum(rowmin, rm)
        cs = jnp.sum(jnp.min(d, axis=0))
        g2p_sum = cs if g2p_sum is None else g2p_sum + cs
    p2g_b = jnp.sqrt(jnp.mean(rowmin)).reshape(1, 1) / bs
    g2p_b = jnp.sqrt(g2p_sum / m).reshape(1, 1) / bs

    @pl.when(b == 0)
    def _init():
        p2g_ref[0] = p2g_b
        g2p_ref[0] = g2p_b

    @pl.when(b > 0)
    def _acc():
        p2g_ref[0] += p2g_b
        g2p_ref[0] += g2p_b

    @pl.when(b == bs - 1)
    def _fin():
        loss_ref[0] = (p2g_ref[0] + g2p_ref[0]) * 0.5


def kernel(points, gts):
    bs, n, _ = points.shape
    m = gts.shape[1]
    loss, p2g, g2p = pl.pallas_call(
        _chamfer_body,
        grid=(bs,),
        in_specs=[
            pl.BlockSpec((1, n, 3), lambda b: (b, 0, 0)),
            pl.BlockSpec((1, m, 3), lambda b: (b, 0, 0)),
        ],
        out_specs=[
            pl.BlockSpec((1, 1, 1), lambda b: (0, 0, 0)),
            pl.BlockSpec((1, 1, 1), lambda b: (0, 0, 0)),
            pl.BlockSpec((1, 1, 1), lambda b: (0, 0, 0)),
        ],
        out_shape=[
            jax.ShapeDtypeStruct((1, 1, 1), jnp.float32),
            jax.ShapeDtypeStruct((1, 1, 1), jnp.float32),
            jax.ShapeDtypeStruct((1, 1, 1), jnp.float32),
        ],
    )(points, gts)
    return (loss.reshape(()), p2g.reshape(()), g2p.reshape(()))


# speedup vs baseline: 1.0586x; 1.0586x over previous
import jax
import jax.numpy as jnp
from jax.experimental import pallas as pl

_CHUNK = 512
_BPG = 2


def _chamfer_body(p_ref, g_ref, loss_ref, p2g_ref, g2p_ref):
    step = pl.program_id(0)
    nsteps = pl.num_programs(0)
    bs = nsteps * _BPG
    p2g_acc = None
    g2p_acc = None
    for bb in range(_BPG):
        pts = p_ref[bb]
        g = g_ref[bb]
        m = g.shape[1]
        px = pts[:, 0:1]
        py = pts[:, 1:2]
        pz = pts[:, 2:3]
        rowmin = None
        g2p_sum = None
        for k in range(0, m, _CHUNK):
            gx = g[0:1, k:k + _CHUNK]
            gy = g[1:2, k:k + _CHUNK]
            gz = g[2:3, k:k + _CHUNK]
            dx = px - gx
            dy = py - gy
            dz = pz - gz
            d = dx * dx + dy * dy + dz * dz
            rm = jnp.min(d, axis=1, keepdims=True)
            rowmin = rm if rowmin is None else jnp.minimum(rowmin, rm)
            cs = jnp.sum(jnp.min(d, axis=0))
            g2p_sum = cs if g2p_sum is None else g2p_sum + cs
        p2g_b = jnp.sqrt(jnp.mean(rowmin)).reshape(1, 1) / bs
        g2p_b = jnp.sqrt(g2p_sum / m).reshape(1, 1) / bs
        p2g_acc = p2g_b if p2g_acc is None else p2g_acc + p2g_b
        g2p_acc = g2p_b if g2p_acc is None else g2p_acc + g2p_b

    @pl.when(step == 0)
    def _init():
        p2g_ref[0] = p2g_acc
        g2p_ref[0] = g2p_acc

    @pl.when(step > 0)
    def _acc():
        p2g_ref[0] += p2g_acc
        g2p_ref[0] += g2p_acc

    @pl.when(step == nsteps - 1)
    def _fin():
        loss_ref[0] = (p2g_ref[0] + g2p_ref[0]) * 0.5


def kernel(points, gts):
    bs, n, _ = points.shape
    m = gts.shape[1]
    gts_t = jnp.transpose(gts, (0, 2, 1))
    loss, p2g, g2p = pl.pallas_call(
        _chamfer_body,
        grid=(bs // _BPG,),
        in_specs=[
            pl.BlockSpec((_BPG, n, 3), lambda b: (b, 0, 0)),
            pl.BlockSpec((_BPG, 3, m), lambda b: (b, 0, 0)),
        ],
        out_specs=[
            pl.BlockSpec((1, 1, 1), lambda b: (0, 0, 0)),
            pl.BlockSpec((1, 1, 1), lambda b: (0, 0, 0)),
            pl.BlockSpec((1, 1, 1), lambda b: (0, 0, 0)),
        ],
        out_shape=[
            jax.ShapeDtypeStruct((1, 1, 1), jnp.float32),
            jax.ShapeDtypeStruct((1, 1, 1), jnp.float32),
            jax.ShapeDtypeStruct((1, 1, 1), jnp.float32),
        ],
    )(points, gts_t)
    return (loss.reshape(()), p2g.reshape(()), g2p.reshape(()))


# speedup vs baseline: 1.0813x; 1.0215x over previous
import jax
import jax.numpy as jnp
from jax.experimental import pallas as pl

_CHUNK = 512


def _chamfer_body(p_ref, g_ref, loss_ref, p2g_ref, g2p_ref):
    b = pl.program_id(0)
    bs = pl.num_programs(0)
    pts = p_ref[0]
    g = g_ref[0]
    m = g.shape[1]
    px = pts[:, 0:1]
    py = pts[:, 1:2]
    pz = pts[:, 2:3]
    rowmin = None
    g2p_sum = None
    for k in range(0, m, _CHUNK):
        gx = g[0:1, k:k + _CHUNK]
        gy = g[1:2, k:k + _CHUNK]
        gz = g[2:3, k:k + _CHUNK]
        dx = px - gx
        dy = py - gy
        dz = pz - gz
        d = dx * dx + dy * dy + dz * dz
        rm = jnp.min(d, axis=1, keepdims=True)
        rowmin = rm if rowmin is None else jnp.minimum(rowmin, rm)
        cs = jnp.sum(jnp.min(d, axis=0))
        g2p_sum = cs if g2p_sum is None else g2p_sum + cs
    p2g_b = jnp.sqrt(jnp.mean(rowmin)).reshape(1, 1) / bs
    g2p_b = jnp.sqrt(g2p_sum / m).reshape(1, 1) / bs

    @pl.when(b == 0)
    def _init():
        p2g_ref[0] = p2g_b
        g2p_ref[0] = g2p_b

    @pl.when(b > 0)
    def _acc():
        p2g_ref[0] += p2g_b
        g2p_ref[0] += g2p_b

    @pl.when(b == bs - 1)
    def _fin():
        loss_ref[0] = (p2g_ref[0] + g2p_ref[0]) * 0.5


def kernel(points, gts):
    bs, n, _ = points.shape
    m = gts.shape[1]
    gts_t = jnp.transpose(gts, (0, 2, 1))
    loss, p2g, g2p = pl.pallas_call(
        _chamfer_body,
        grid=(bs,),
        in_specs=[
            pl.BlockSpec((1, n, 3), lambda b: (b, 0, 0)),
            pl.BlockSpec((1, 3, m), lambda b: (b, 0, 0)),
        ],
        out_specs=[
            pl.BlockSpec((1, 1, 1), lambda b: (0, 0, 0)),
            pl.BlockSpec((1, 1, 1), lambda b: (0, 0, 0)),
            pl.BlockSpec((1, 1, 1), lambda b: (0, 0, 0)),
        ],
        out_shape=[
            jax.ShapeDtypeStruct((1, 1, 1), jnp.float32),
            jax.ShapeDtypeStruct((1, 1, 1), jnp.float32),
            jax.ShapeDtypeStruct((1, 1, 1), jnp.float32),
        ],
    )(points, gts_t)
    return (loss.reshape(()), p2g.reshape(()), g2p.reshape(()))
